# baseline (speedup 1.0000x reference)
import functools

import jax
import jax.numpy as jnp
from jax import lax
from jax.experimental import pallas as pl
from jax.experimental.pallas import tpu as pltpu
from jax.experimental.pallas import tpu_sc as plsc

EMB = 64
WIN = 5
NC = 2
NS = 16
NW = NC * NS
FPC = 16
CHUNK = FPC * WIN


def _pack_table(table, S, br):
    V = table.shape[0]
    R = V - S
    nblk = S // br
    ntail = V - 2 * S
    assert nblk * br == S and 0 < ntail <= br and R == S + ntail
    assert S % 128 == 0 and br % 128 == 0
    tT = table.T
    tail = jnp.pad(table[2 * S:, :], ((0, 0), (EMB, 0)))

    def fetch(t_ref, col0, xbuf, sem):
        return pltpu.make_async_copy(t_ref.at[:, pl.ds(col0, br)], xbuf, sem)

    def body(t_ref, tail_ref, o_ref, xbL, xbR, sem):
        i = pl.program_id(0)

        @pl.when(i == 0)
        def _prologue():
            fetch(t_ref, 0, xbL[0], sem).start()
            fetch(t_ref, S, xbR[0], sem).start()

        @pl.when(i < nblk)
        def _main_blocks():
            for b in range(2):
                @pl.when(lax.rem(i, 2) == b)
                def _step(b=b):
                    fetch(t_ref, i * br, xbL[b], sem).wait()
                    fetch(t_ref, S + i * br, xbR[b], sem).wait()

                    @pl.when(i + 1 < nblk)
                    def _prefetch(b=b):
                        fetch(t_ref, (i + 1) * br, xbL[1 - b], sem).start()
                        fetch(t_ref, S + (i + 1) * br, xbR[1 - b], sem).start()

                    o_ref[:, :EMB] = lax.transpose(xbL[b][...], (1, 0))
                    o_ref[:, EMB:] = lax.transpose(xbR[b][...], (1, 0))

        @pl.when(i == nblk)
        def _tail_block():
            o_ref[pl.ds(0, ntail), :] = tail_ref[...]

    return pl.pallas_call(
        body,
        grid=(nblk + 1,),
        in_specs=[
            pl.BlockSpec(memory_space=pl.ANY),
            pl.BlockSpec((ntail, 2 * EMB), lambda i: (0, 0)),
        ],
        out_specs=pl.BlockSpec((br, 2 * EMB), lambda i: (i, 0)),
        out_shape=jax.ShapeDtypeStruct((R, 2 * EMB), jnp.float32),
        scratch_shapes=[
            [pltpu.VMEM((EMB, br), jnp.float32) for _ in range(2)],
            [pltpu.VMEM((EMB, br), jnp.float32) for _ in range(2)],
            pltpu.SemaphoreType.DMA,
        ],
    )(tT, tail)


def _sc_gather_sum(idx_list, tab_list, S_list, out_cols):
    n = len(idx_list)
    total = idx_list[0].shape[0]
    per_w = total // NW
    n_chunks = per_w // CHUNK
    assert per_w % CHUNK == 0 and n_chunks % 2 == 0
    frows_w = per_w // WIN
    n_flat = total // WIN

    mesh = plsc.VectorSubcoreMesh(
        core_axis_name="c", subcore_axis_name="s", num_cores=NC, num_subcores=NS
    )

    def core(idx_hbm, tab_hbm, out_hbm, iv, off_v, bufs, out_v, gsem, wsem):
        wid = lax.axis_index("s") * NC + lax.axis_index("c")
        base = wid * per_w
        frow0 = wid * frows_w

        for t in range(n):
            pltpu.sync_copy(idx_hbm[t].at[pl.ds(base, per_w)], iv[t])

        def xform(j, carry):
            sl = pl.ds(j * 16, 16)
            for t in range(n):
                v = iv[t][sl]
                big = v >= S_list[t]
                off_v[t][sl] = jnp.where(big, 64, 0)
                iv[t][sl] = jnp.where(big, v - S_list[t], v)
            return carry

        lax.fori_loop(0, per_w // 16, xform, 0)

        zeros = jnp.zeros((16,), jnp.float32)
        for ov in out_v:
            for f in range(FPC):
                for c in range(WIN * EMB, out_cols, 16):
                    ov[f, pl.ds(c, 16)] = zeros

        def gathers(c, b):
            sl = pl.ds(c * CHUNK, CHUNK)
            return tuple(
                pltpu.make_async_copy(tab_hbm[t].at[iv[t].at[sl]], bufs[t][b], gsem)
                for t in range(n)
            )

        for d in gathers(0, 0):
            d.start()

        def compute(c, b):
            ovecs = [
                [off_v[t][pl.ds(c * CHUNK + k * 16, 16)]
                 for k in range(CHUNK // 16)]
                for t in range(n)
            ]
            for f in range(FPC):
                for w in range(WIN):
                    g = f * WIN + w
                    hs = [
                        pl.multiple_of(ovecs[t][g // 16][g % 16], 64)
                        for t in range(n)
                    ]
                    for cc in range(EMB // 16):
                        acc = bufs[0][b][g, pl.ds(hs[0] + cc * 16, 16)]
                        for t in range(1, n):
                            acc = acc + bufs[t][b][g, pl.ds(hs[t] + cc * 16, 16)]
                        out_v[b][f, pl.ds(w * EMB + cc * 16, 16)] = acc

        def pair_body(i, carry):
            for b in range(2):
                c = i * 2 + b
                for d in gathers(c, b):
                    d.wait()

                @pl.when(c + 1 < n_chunks)
                def _start_next(b=b, c=c):
                    for d in gathers(c + 1, 1 - b):
                        d.start()

                @pl.when(c >= 2)
                def _drain_prev(b=b, c=c):
                    pltpu.make_async_copy(
                        out_v[b], out_hbm.at[pl.ds(frow0 + c * FPC, FPC)], wsem
                    ).wait()

                compute(c, b)
                pltpu.make_async_copy(
                    out_v[b], out_hbm.at[pl.ds(frow0 + c * FPC, FPC)], wsem
                ).start()
            return carry

        lax.fori_loop(0, n_chunks // 2, pair_body, 0)
        pltpu.make_async_copy(
            out_v[0], out_hbm.at[pl.ds(frow0, FPC)], wsem).wait()
        pltpu.make_async_copy(
            out_v[1], out_hbm.at[pl.ds(frow0, FPC)], wsem).wait()

    if n == 2:
        def body(i0, i1, t0, t1, out_hbm, v0, v1, o0, o1, b0, b1,
                 out_v, gsem, wsem):
            core([i0, i1], [t0, t1], out_hbm, [v0, v1], [o0, o1], [b0, b1],
                 out_v, gsem, wsem)
    else:
        def body(i0, t0, out_hbm, v0, o0, b0, out_v, gsem, wsem):
            core([i0], [t0], out_hbm, [v0], [o0], [b0], out_v, gsem, wsem)

    k = pl.kernel(
        body,
        out_type=jax.ShapeDtypeStruct((n_flat, out_cols), jnp.float32),
        mesh=mesh,
        compiler_params=pltpu.CompilerParams(needs_layout_passes=False),
        scratch_types=(
            [pltpu.VMEM((per_w,), jnp.int32) for _ in range(n)]
            + [pltpu.VMEM((per_w,), jnp.int32) for _ in range(n)]
            + [[pltpu.VMEM((CHUNK, 128), jnp.float32) for _ in range(2)]
               for _ in range(n)]
            + [[pltpu.VMEM((FPC, out_cols), jnp.float32) for _ in range(2)],
               pltpu.SemaphoreType.DMA,
               pltpu.SemaphoreType.DMA]
        ),
    )
    return k(*(list(idx_list) + list(tab_list)))


def _mlp(part_a, part_b, W1, b1, W2, b2):
    B, KP = part_a.shape
    K = W1.shape[0]
    H = W1.shape[1]
    T = W2.shape[1]
    BM = 1024
    assert B % BM == 0

    def body(a_ref, b_ref, w1_ref, b1_ref, w2_ref, b2_ref, out_ref):
        x = a_ref[...][:, :K] + b_ref[...][:, :K]
        h = jnp.tanh(
            jnp.dot(x, w1_ref[...], preferred_element_type=jnp.float32)
            + b1_ref[...]
        )
        out_ref[...] = (
            jnp.dot(h, w2_ref[...], preferred_element_type=jnp.float32) + b2_ref[...]
        )

    return pl.pallas_call(
        body,
        grid=(B // BM,),
        in_specs=[
            pl.BlockSpec((BM, KP), lambda i: (i, 0)),
            pl.BlockSpec((BM, KP), lambda i: (i, 0)),
            pl.BlockSpec((K, H), lambda i: (0, 0)),
            pl.BlockSpec((1, H), lambda i: (0, 0)),
            pl.BlockSpec((H, T), lambda i: (0, 0)),
            pl.BlockSpec((1, T), lambda i: (0, 0)),
        ],
        out_specs=pl.BlockSpec((BM, T), lambda i: (i, 0)),
        out_shape=jax.ShapeDtypeStruct((B, T), jnp.float32),
    )(part_a, part_b, W1, b1.reshape(1, H), W2, b2.reshape(1, T))


def kernel(xw, xp, xs, Ww, Wp, Ws, W1, b1, W2, b2):
    W2p = _pack_table(Wp, 49920, 9984)
    W2s = _pack_table(Ws, 49920, 9984)
    ps384 = _sc_gather_sum(
        [xp.reshape(-1), xs.reshape(-1)], [W2p, W2s], [49920, 49920], 384
    )
    W2w = _pack_table(Ww, 499968, 11904)
    w384 = _sc_gather_sum([xw.reshape(-1)], [W2w], [499968], 384)
    return _mlp(ps384, w384, W1, b1, W2, b2)


# speedup vs baseline: 1.1048x; 1.1048x over previous
import functools

import jax
import jax.numpy as jnp
from jax import lax
from jax.experimental import pallas as pl
from jax.experimental.pallas import tpu as pltpu
from jax.experimental.pallas import tpu_sc as plsc

EMB = 64
WIN = 5
NC = 2
NS = 16
NW = NC * NS
FPC = 16
CHUNK = FPC * WIN


def _pack_table(table, br):
    V = table.shape[0]
    main = (V // 128) * 128
    ntail = V - main
    assert main % br == 0 and 0 < ntail < br
    nblk = main // br
    tT = table.T
    tail = jnp.pad(table[main:, :], ((0, 0), (0, 2 * EMB - EMB)))

    def fetch(t_ref, i, xbuf, sem):
        return pltpu.make_async_copy(t_ref.at[:, pl.ds(i * br, br)], xbuf, sem)

    def body(t_ref, tail_ref, o_ref, xb, sem):
        i = pl.program_id(0)

        @pl.when(i == 0)
        def _prologue():
            fetch(t_ref, 0, xb[0], sem).start()

        @pl.when(i < nblk)
        def _main_blocks():
            for b in range(2):
                @pl.when(lax.rem(i, 2) == b)
                def _step(b=b):
                    fetch(t_ref, i, xb[b], sem).wait()

                    @pl.when(i + 1 < nblk)
                    def _prefetch(b=b):
                        fetch(t_ref, i + 1, xb[1 - b], sem).start()

                    o_ref[:, :EMB] = lax.transpose(xb[b][...], (1, 0))

        @pl.when(i == nblk)
        def _tail_block():
            o_ref[pl.ds(0, 128), :] = jnp.concatenate(
                [tail_ref[...]] + [tail_ref[...]] * ((128 - ntail) // ntail),
                axis=0,
            )[:128]

    return pl.pallas_call(
        body,
        grid=(nblk + 1,),
        in_specs=[
            pl.BlockSpec(memory_space=pl.ANY),
            pl.BlockSpec((ntail, 2 * EMB), lambda i: (0, 0)),
        ],
        out_specs=pl.BlockSpec((br, 2 * EMB), lambda i: (i, 0)),
        out_shape=jax.ShapeDtypeStruct((V, 2 * EMB), jnp.float32),
        scratch_shapes=[
            [pltpu.VMEM((EMB, br), jnp.float32) for _ in range(2)],
            pltpu.SemaphoreType.DMA,
        ],
    )(tT, tail)


def _sc_gather_sum(idx_list, tab_list, out_cols):
    n = len(idx_list)
    total = idx_list[0].shape[0]
    per_w = total // NW
    n_chunks = per_w // CHUNK
    assert per_w % CHUNK == 0 and n_chunks % 2 == 0
    frows_w = per_w // WIN
    n_flat = total // WIN

    mesh = plsc.VectorSubcoreMesh(
        core_axis_name="c", subcore_axis_name="s", num_cores=NC, num_subcores=NS
    )

    def core(idx_hbm, tab_hbm, out_hbm, iv, bufs, out_v, gsem, wsem):
        wid = lax.axis_index("s") * NC + lax.axis_index("c")
        base = wid * per_w
        frow0 = wid * frows_w

        for t in range(n):
            pltpu.sync_copy(idx_hbm[t].at[pl.ds(base, per_w)], iv[t])

        zeros = jnp.zeros((16,), jnp.float32)
        for ov in out_v:
            for f in range(FPC):
                for c in range(WIN * EMB, out_cols, 16):
                    ov[f, pl.ds(c, 16)] = zeros

        def gathers(c, b):
            sl = pl.ds(c * CHUNK, CHUNK)
            return tuple(
                pltpu.make_async_copy(tab_hbm[t].at[iv[t].at[sl]], bufs[t][b], gsem)
                for t in range(n)
            )

        for d in gathers(0, 0):
            d.start()

        def compute(c, b):
            for f in range(FPC):
                for w in range(WIN):
                    g = f * WIN + w
                    for cc in range(EMB // 16):
                        acc = bufs[0][b][g, pl.ds(cc * 16, 16)]
                        for t in range(1, n):
                            acc = acc + bufs[t][b][g, pl.ds(cc * 16, 16)]
                        out_v[b][f, pl.ds(w * EMB + cc * 16, 16)] = acc

        def pair_body(i, carry):
            for b in range(2):
                c = i * 2 + b
                for d in gathers(c, b):
                    d.wait()

                @pl.when(c + 1 < n_chunks)
                def _start_next(b=b, c=c):
                    for d in gathers(c + 1, 1 - b):
                        d.start()

                @pl.when(c >= 2)
                def _drain_prev(b=b, c=c):
                    pltpu.make_async_copy(
                        out_v[b], out_hbm.at[pl.ds(frow0 + c * FPC, FPC)], wsem
                    ).wait()

                compute(c, b)
                pltpu.make_async_copy(
                    out_v[b], out_hbm.at[pl.ds(frow0 + c * FPC, FPC)], wsem
                ).start()
            return carry

        lax.fori_loop(0, n_chunks // 2, pair_body, 0)
        pltpu.make_async_copy(
            out_v[0], out_hbm.at[pl.ds(frow0, FPC)], wsem).wait()
        pltpu.make_async_copy(
            out_v[1], out_hbm.at[pl.ds(frow0, FPC)], wsem).wait()

    if n == 2:
        def body(i0, i1, t0, t1, out_hbm, v0, v1, b0, b1, out_v, gsem, wsem):
            core([i0, i1], [t0, t1], out_hbm, [v0, v1], [b0, b1],
                 out_v, gsem, wsem)
    else:
        def body(i0, t0, out_hbm, v0, b0, out_v, gsem, wsem):
            core([i0], [t0], out_hbm, [v0], [b0], out_v, gsem, wsem)

    k = pl.kernel(
        body,
        out_type=jax.ShapeDtypeStruct((n_flat, out_cols), jnp.float32),
        mesh=mesh,
        compiler_params=pltpu.CompilerParams(needs_layout_passes=False),
        scratch_types=(
            [pltpu.VMEM((per_w,), jnp.int32) for _ in range(n)]
            + [[pltpu.VMEM((CHUNK, 128), jnp.float32) for _ in range(2)]
               for _ in range(n)]
            + [[pltpu.VMEM((FPC, out_cols), jnp.float32) for _ in range(2)],
               pltpu.SemaphoreType.DMA,
               pltpu.SemaphoreType.DMA]
        ),
    )
    return k(*(list(idx_list) + list(tab_list)))


def _mlp(part_a, part_b, W1, b1, W2, b2):
    B, KP = part_a.shape
    K = W1.shape[0]
    H = W1.shape[1]
    T = W2.shape[1]
    BM = 1024
    assert B % BM == 0

    def body(a_ref, b_ref, w1_ref, b1_ref, w2_ref, b2_ref, out_ref):
        x = a_ref[...][:, :K] + b_ref[...][:, :K]
        h = jnp.tanh(
            jnp.dot(x, w1_ref[...], preferred_element_type=jnp.float32)
            + b1_ref[...]
        )
        out_ref[...] = (
            jnp.dot(h, w2_ref[...], preferred_element_type=jnp.float32) + b2_ref[...]
        )

    return pl.pallas_call(
        body,
        grid=(B // BM,),
        in_specs=[
            pl.BlockSpec((BM, KP), lambda i: (i, 0)),
            pl.BlockSpec((BM, KP), lambda i: (i, 0)),
            pl.BlockSpec((K, H), lambda i: (0, 0)),
            pl.BlockSpec((1, H), lambda i: (0, 0)),
            pl.BlockSpec((H, T), lambda i: (0, 0)),
            pl.BlockSpec((1, T), lambda i: (0, 0)),
        ],
        out_specs=pl.BlockSpec((BM, T), lambda i: (i, 0)),
        out_shape=jax.ShapeDtypeStruct((B, T), jnp.float32),
    )(part_a, part_b, W1, b1.reshape(1, H), W2, b2.reshape(1, T))


def kernel(xw, xp, xs, Ww, Wp, Ws, W1, b1, W2, b2):
    W2p = _pack_table(Wp, 9088)
    W2s = _pack_table(Ws, 9088)
    ps384 = _sc_gather_sum([xp.reshape(-1), xs.reshape(-1)], [W2p, W2s], 384)
    W2w = _pack_table(Ww, 15872)
    w384 = _sc_gather_sum([xw.reshape(-1)], [W2w], 384)
    return _mlp(ps384, w384, W1, b1, W2, b2)
